# baseline (speedup 1.0000x reference)
import jax
import jax.numpy as jnp
from jax import lax
from jax.experimental import pallas as pl
from jax.experimental.pallas import tpu as pltpu
from jax.experimental.pallas import tpu_sc as plsc

N = 10000
E = 320000
D = 128

NC = 2
NS = 16
NW = NC * NS
PER_W = E // NW
CH = 40
NCHUNK = PER_W // CH
ROWS_S = 624
TAIL_0 = ROWS_S * NS
TAIL_N = N - TAIL_0
CPY = 24
NCPY = ROWS_S // CPY


def _act(x):
    return jnp.where(x >= 0, x, 0.01 * x)


def _pre_body(nf, gf, wn, bn, wg, bg, w1, w2, w4, h_o, u_o, a_o, b_o):
    h = _act(jnp.dot(nf[...], wn[...], preferred_element_type=jnp.float32) + bn[...])
    u = _act(jnp.dot(gf[...], wg[...], preferred_element_type=jnp.float32) + bg[...])
    h_o[...] = h
    u_o[...] = u
    a_o[...] = jnp.dot(h, w1[...], preferred_element_type=jnp.float32)
    b_o[...] = (jnp.dot(h, w2[...], preferred_element_type=jnp.float32)
                + jnp.dot(u, w4[...], preferred_element_type=jnp.float32))


def _pre_call(nf, gf, wn, bn, wg, bg, w1, w2, w4):
    BN = 2000
    grid = (N // BN,)
    row = lambda i: (i, 0)
    full = lambda i: (0, 0)
    spec_r = pl.BlockSpec((BN, D), row)
    spec_w = pl.BlockSpec((D, D), full)
    spec_b = pl.BlockSpec((1, D), full)
    return pl.pallas_call(
        _pre_body,
        grid=grid,
        in_specs=[spec_r, spec_r, spec_w, spec_b, spec_w, spec_b,
                  spec_w, spec_w, spec_w],
        out_specs=[spec_r, spec_r, spec_r, spec_r],
        out_shape=[jax.ShapeDtypeStruct((N, D), jnp.float32)] * 4,
    )(nf, gf, wn, bn, wg, bg, w1, w2, w4)


def _edgec_body(ef, we, be, w3, beu, c_o):
    f = _act(jnp.dot(ef[...], we[...], preferred_element_type=jnp.float32) + be[...])
    c_o[...] = jnp.dot(f, w3[...], preferred_element_type=jnp.float32) + beu[...]


def _edgec_call(ef, we, be, w3, beu):
    BE = 4000
    grid = (E // BE,)
    row = lambda i: (i, 0)
    full = lambda i: (0, 0)
    return pl.pallas_call(
        _edgec_body,
        grid=grid,
        in_specs=[pl.BlockSpec((BE, D), row), pl.BlockSpec((D, D), full),
                  pl.BlockSpec((1, D), full), pl.BlockSpec((D, D), full),
                  pl.BlockSpec((1, D), full)],
        out_specs=pl.BlockSpec((BE, D), row),
        out_shape=jax.ShapeDtypeStruct((E, D), jnp.float32),
    )(ef, we, be, w3, beu)


G = 10
NGRP = NCHUNK // G
HALF = G // 2


def _sc_body(a_hbm, b_hbm, c_hbm, src3, dst3,
             fnew_hbm, aggp_hbm, degp_hbm,
             idxs_buf, idxd_buf,
             buf_a0, buf_a1, buf_b0, buf_b1, buf_c0,
             buf_ones,
             agg_sh, deg_sh,
             sa0, sa1, sb0, sb1):
    cbuf = buf_a0.at[pl.ds(0, CPY)]
    dbuf = buf_ones.at[pl.ds(0, CPY)]
    c = lax.axis_index("c")
    s = lax.axis_index("s")
    wid = s * NC + c

    zero16 = jnp.zeros((16,), jnp.float32)
    onehot = jnp.where(lax.iota(jnp.int32, 16) == 0, 1.0, 0.0).astype(jnp.float32)

    def zrow(i, carry):
        for k in range(D // 16):
            buf_a0[i, pl.ds(k * 16, 16)] = zero16
        buf_ones[i, :] = zero16
        return carry

    lax.fori_loop(0, CPY, zrow, 0)

    for t in range(NCPY):
        off = s * ROWS_S + t * CPY
        pltpu.sync_copy(cbuf, agg_sh.at[pl.ds(off, CPY)])
        pltpu.sync_copy(dbuf, deg_sh.at[pl.ds(off, CPY)])

    @pl.when(s == NS - 1)
    def _():
        pltpu.sync_copy(cbuf.at[pl.ds(0, TAIL_N)], agg_sh.at[pl.ds(TAIL_0, TAIL_N)])
        pltpu.sync_copy(dbuf.at[pl.ds(0, TAIL_N)], deg_sh.at[pl.ds(TAIL_0, TAIL_N)])

    plsc.subcore_barrier()

    def orow(i, carry):
        buf_ones[i, :] = onehot
        return carry

    lax.fori_loop(0, CH, orow, 0)

    BUFS0 = (buf_a0, buf_b0, sa0, sb0)
    BUFS1 = (buf_a1, buf_b1, sa1, sb1)

    def fire(j, bufs):
        ba, bb, sa, sb = bufs
        pltpu.async_copy(a_hbm.at[idxs_buf.at[j]], ba, sa)
        pltpu.async_copy(b_hbm.at[idxd_buf.at[j]], bb, sb)

    def process(gbase, j, bufs):
        ba, bb, sa, sb = bufs
        sl = pl.ds(gbase + j * CH, CH)
        pltpu.sync_copy(c_hbm.at[sl], buf_c0)
        pltpu.make_async_copy(a_hbm.at[idxs_buf.at[j]], ba, sa).wait()
        pltpu.make_async_copy(b_hbm.at[idxd_buf.at[j]], bb, sb).wait()

        def row(i, carry2):
            for k in range(D // 16):
                ks = pl.ds(k * 16, 16)
                x = ba[i, ks] + bb[i, ks] + buf_c0[i, ks]
                buf_c0[i, ks] = jnp.where(x >= 0, x, 0.01 * x)
            return carry2

        lax.fori_loop(0, CH, row, 0)
        pltpu.sync_copy(buf_c0, agg_sh.at[idxd_buf.at[j]], add=True)
        pltpu.sync_copy(buf_ones, deg_sh.at[idxd_buf.at[j]], add=True)
        pltpu.sync_copy(buf_c0, fnew_hbm.at[sl])

    def group(g, carry):
        gsl = pl.ds(g * G, G)
        pltpu.sync_copy(src3.at[wid, gsl], idxs_buf)
        pltpu.sync_copy(dst3.at[wid, gsl], idxd_buf)
        gbase = wid * PER_W + g * (G * CH)
        fire(0, BUFS0)

        def pair(t, carry2):
            fire(2 * t + 1, BUFS1)
            process(gbase, 2 * t, BUFS0)

            @pl.when(t + 1 < HALF)
            def _():
                fire(2 * t + 2, BUFS0)

            process(gbase, 2 * t + 1, BUFS1)
            return carry2

        lax.fori_loop(0, HALF, pair, 0)
        return carry

    lax.fori_loop(0, NGRP, group, 0)
    plsc.subcore_barrier()

    for t in range(NCPY):
        off = s * ROWS_S + t * CPY
        pltpu.sync_copy(agg_sh.at[pl.ds(off, CPY)], cbuf)
        pltpu.sync_copy(cbuf, aggp_hbm.at[c, pl.ds(off, CPY)])
        pltpu.sync_copy(deg_sh.at[pl.ds(off, CPY)], dbuf)
        pltpu.sync_copy(dbuf, degp_hbm.at[c, pl.ds(off, CPY)])

    @pl.when(s == NS - 1)
    def _():
        tl = pl.ds(TAIL_0, TAIL_N)
        pltpu.sync_copy(agg_sh.at[tl], cbuf.at[pl.ds(0, TAIL_N)])
        pltpu.sync_copy(cbuf.at[pl.ds(0, TAIL_N)], aggp_hbm.at[c, tl])
        pltpu.sync_copy(deg_sh.at[tl], dbuf.at[pl.ds(0, TAIL_N)])
        pltpu.sync_copy(dbuf.at[pl.ds(0, TAIL_N)], degp_hbm.at[c, tl])


def _sc_call(a_t, b_t, c_t, src, dst):
    mesh = plsc.VectorSubcoreMesh(core_axis_name="c", subcore_axis_name="s")
    src3 = src.reshape(NW, NCHUNK, CH)
    dst3 = dst.reshape(NW, NCHUNK, CH)
    fn = pl.kernel(
        _sc_body,
        out_type=[jax.ShapeDtypeStruct((E, D), jnp.float32),
                  jax.ShapeDtypeStruct((NC, N, D), jnp.float32),
                  jax.ShapeDtypeStruct((NC, N, 16), jnp.float32)],
        mesh=mesh,
        compiler_params=pltpu.CompilerParams(use_tc_tiling_on_sc=False),
        scratch_types=[
            pltpu.VMEM((G, CH), jnp.int32),
            pltpu.VMEM((G, CH), jnp.int32),
            pltpu.VMEM((CH, D), jnp.float32),
            pltpu.VMEM((CH, D), jnp.float32),
            pltpu.VMEM((CH, D), jnp.float32),
            pltpu.VMEM((CH, D), jnp.float32),
            pltpu.VMEM((CH, D), jnp.float32),
            pltpu.VMEM((CH, 16), jnp.float32),
            pltpu.VMEM_SHARED((N, D), jnp.float32),
            pltpu.VMEM_SHARED((N, 16), jnp.float32),
            pltpu.SemaphoreType.DMA,
            pltpu.SemaphoreType.DMA,
            pltpu.SemaphoreType.DMA,
            pltpu.SemaphoreType.DMA,
        ],
    )
    return fn(a_t, b_t, c_t, src3, dst3)


def _eres_body(fn_in, ef, out):
    out[...] = fn_in[...] + ef[...]


def _eres_call(f_new, ef):
    BE = 4000
    row = lambda i: (i, 0)
    return pl.pallas_call(
        _eres_body,
        grid=(E // BE,),
        in_specs=[pl.BlockSpec((BE, D), row), pl.BlockSpec((BE, D), row)],
        out_specs=pl.BlockSpec((BE, D), row),
        out_shape=jax.ShapeDtypeStruct((E, D), jnp.float32),
    )(f_new, ef)


def _node_body(h, u, aggp, degp, nf, gf, v1, v2, v3, bnu,
               nout, g8, acc):
    i = pl.program_id(0)

    @pl.when(i == 0)
    def _():
        acc[...] = jnp.zeros_like(acc)

    agg = aggp[0] + aggp[1]
    deg = jnp.maximum(degp[0, :, 0] + degp[1, :, 0], 1.0)
    hf = agg / deg[:, None]
    node_new = _act(jnp.dot(h[...], v1[...], preferred_element_type=jnp.float32)
                    + jnp.dot(hf, v2[...], preferred_element_type=jnp.float32)
                    + jnp.dot(u[...], v3[...], preferred_element_type=jnp.float32)
                    + bnu[...])
    nout[...] = node_new + nf[...]
    acc[0:1, :] += jnp.sum(node_new, axis=0, keepdims=True)
    acc[1:2, :] += jnp.sum(agg, axis=0, keepdims=True)
    acc[2:3, :] += jnp.sum(gf[...], axis=0, keepdims=True)
    npool = acc[0:1, :] / N
    epool = acc[1:2, :] / E
    gpool = acc[2:3, :] / N
    g = _act(jnp.dot(npool, v1[...], preferred_element_type=jnp.float32)
             + jnp.dot(epool, v2[...], preferred_element_type=jnp.float32)
             + jnp.dot(gpool, v3[...], preferred_element_type=jnp.float32)
             + bnu[...])
    g8[...] = jnp.broadcast_to(g, (8, D))


def _node_call(h, u, aggp, degp, nf, gf, v1, v2, v3, bnu):
    BN = 1000
    grid = (N // BN,)
    row = lambda i: (i, 0)
    full = lambda i: (0, 0)
    return pl.pallas_call(
        _node_body,
        grid=grid,
        in_specs=[pl.BlockSpec((BN, D), row), pl.BlockSpec((BN, D), row),
                  pl.BlockSpec((NC, BN, D), lambda i: (0, i, 0)),
                  pl.BlockSpec((NC, BN, 16), lambda i: (0, i, 0)),
                  pl.BlockSpec((BN, D), row), pl.BlockSpec((BN, D), row),
                  pl.BlockSpec((D, D), full), pl.BlockSpec((D, D), full),
                  pl.BlockSpec((D, D), full), pl.BlockSpec((1, D), full)],
        out_specs=[pl.BlockSpec((BN, D), row), pl.BlockSpec((8, D), full)],
        out_shape=[jax.ShapeDtypeStruct((N, D), jnp.float32),
                   jax.ShapeDtypeStruct((8, D), jnp.float32)],
        scratch_shapes=[pltpu.VMEM((8, D), jnp.float32)],
    )(h, u, aggp, degp, nf, gf, v1, v2, v3, bnu)


def _gout_body(gf, g8, out):
    out[...] = gf[...] + g8[0:1, :]


def _gout_call(gf, g8):
    BN = 2000
    return pl.pallas_call(
        _gout_body,
        grid=(N // BN,),
        in_specs=[pl.BlockSpec((BN, D), lambda i: (i, 0)),
                  pl.BlockSpec((8, D), lambda i: (0, 0))],
        out_specs=pl.BlockSpec((BN, D), lambda i: (i, 0)),
        out_shape=jax.ShapeDtypeStruct((N, D), jnp.float32),
    )(gf, g8)


def kernel(node_feats, edge_feats, graph_feats, edge_index,
           W_n, b_n, W_e, b_e, W_g, b_g, W_eu, b_eu, W_nu, b_nu):
    src = edge_index[0].astype(jnp.int32)
    dst = edge_index[1].astype(jnp.int32)
    w1, w2, w3, w4 = W_eu[0:D], W_eu[D:2 * D], W_eu[2 * D:3 * D], W_eu[3 * D:4 * D]
    v1, v2, v3 = W_nu[0:D], W_nu[D:2 * D], W_nu[2 * D:3 * D]
    bn = b_n.reshape(1, D)
    be = b_e.reshape(1, D)
    bg = b_g.reshape(1, D)
    beu = b_eu.reshape(1, D)
    bnu = b_nu.reshape(1, D)

    h, u, a_t, b_t = _pre_call(node_feats, graph_feats, W_n, bn, W_g, bg, w1, w2, w4)
    c_t = _edgec_call(edge_feats, W_e, be, w3, beu)
    f_new, aggp, degp = _sc_call(a_t, b_t, c_t, src, dst)
    edge_new = _eres_call(f_new, edge_feats)
    node_new, g8 = _node_call(h, u, aggp, degp, node_feats, graph_feats, v1, v2, v3, bnu)
    g_new = _gout_call(graph_feats, g8)
    return (node_new, edge_new, g_new)


# speedup vs baseline: 1.4485x; 1.4485x over previous
import jax
import jax.numpy as jnp
from jax import lax
from jax.experimental import pallas as pl
from jax.experimental.pallas import tpu as pltpu
from jax.experimental.pallas import tpu_sc as plsc

N = 10000
E = 320000
D = 128

NC = 2
NS = 16
NW = NC * NS
PER_W = E // NW
CH = 40
NCHUNK = PER_W // CH
ROWS_S = 624
TAIL_0 = ROWS_S * NS
TAIL_N = N - TAIL_0
CPY = 24
NCPY = ROWS_S // CPY


def _act(x):
    return jnp.where(x >= 0, x, 0.01 * x)


def _pre_body(nf, gf, wn, bn, wg, bg, w1, w2, w4, h_o, u_o, a_o, b_o):
    h = _act(jnp.dot(nf[...], wn[...], preferred_element_type=jnp.float32) + bn[...])
    u = _act(jnp.dot(gf[...], wg[...], preferred_element_type=jnp.float32) + bg[...])
    h_o[...] = h
    u_o[...] = u
    a_o[...] = jnp.dot(h, w1[...], preferred_element_type=jnp.float32)
    b_o[...] = (jnp.dot(h, w2[...], preferred_element_type=jnp.float32)
                + jnp.dot(u, w4[...], preferred_element_type=jnp.float32))


def _pre_call(nf, gf, wn, bn, wg, bg, w1, w2, w4):
    BN = 2000
    grid = (N // BN,)
    row = lambda i: (i, 0)
    full = lambda i: (0, 0)
    spec_r = pl.BlockSpec((BN, D), row)
    spec_w = pl.BlockSpec((D, D), full)
    spec_b = pl.BlockSpec((1, D), full)
    return pl.pallas_call(
        _pre_body,
        grid=grid,
        in_specs=[spec_r, spec_r, spec_w, spec_b, spec_w, spec_b,
                  spec_w, spec_w, spec_w],
        out_specs=[spec_r, spec_r, spec_r, spec_r],
        out_shape=[jax.ShapeDtypeStruct((N, D), jnp.float32)] * 4,
    )(nf, gf, wn, bn, wg, bg, w1, w2, w4)


def _edgec_body(ef, we, be, w3, beu, c_o):
    f = _act(jnp.dot(ef[...], we[...], preferred_element_type=jnp.float32) + be[...])
    c_o[...] = jnp.dot(f, w3[...], preferred_element_type=jnp.float32) + beu[...]


def _edgec_call(ef, we, be, w3, beu):
    BE = 4000
    grid = (E // BE,)
    row = lambda i: (i, 0)
    full = lambda i: (0, 0)
    return pl.pallas_call(
        _edgec_body,
        grid=grid,
        in_specs=[pl.BlockSpec((BE, D), row), pl.BlockSpec((D, D), full),
                  pl.BlockSpec((1, D), full), pl.BlockSpec((D, D), full),
                  pl.BlockSpec((1, D), full)],
        out_specs=pl.BlockSpec((BE, D), row),
        out_shape=jax.ShapeDtypeStruct((E, D), jnp.float32),
    )(ef, we, be, w3, beu)


G = 50
NGRP = NCHUNK // G
HALF = G // 2


def _sc_body(a_hbm, b_hbm, c_hbm, src3, dst3,
             fnew_hbm, aggp_hbm, degp_hbm,
             idxs_buf, idxd_buf,
             buf_a0, buf_a1, buf_b0, buf_b1, buf_c0, buf_c1,
             buf_ones,
             agg_sh, deg_sh,
             sa0, sa1, sb0, sb1, sc0, sc1, ss0, ss1, so0, so1):
    cbuf = buf_a0.at[pl.ds(0, CPY)]
    dbuf = buf_ones.at[pl.ds(0, CPY)]
    c = lax.axis_index("c")
    s = lax.axis_index("s")
    wid = s * NC + c

    zero16 = jnp.zeros((16,), jnp.float32)
    onehot = jnp.where(lax.iota(jnp.int32, 16) == 0, 1.0, 0.0).astype(jnp.float32)

    def zrow(i, carry):
        for k in range(D // 16):
            buf_a0[i, pl.ds(k * 16, 16)] = zero16
        buf_ones[i, :] = zero16
        return carry

    lax.fori_loop(0, CPY, zrow, 0)

    for t in range(NCPY):
        off = s * ROWS_S + t * CPY
        pltpu.sync_copy(cbuf, agg_sh.at[pl.ds(off, CPY)])
        pltpu.sync_copy(dbuf, deg_sh.at[pl.ds(off, CPY)])

    @pl.when(s == NS - 1)
    def _():
        pltpu.sync_copy(cbuf.at[pl.ds(0, TAIL_N)], agg_sh.at[pl.ds(TAIL_0, TAIL_N)])
        pltpu.sync_copy(dbuf.at[pl.ds(0, TAIL_N)], deg_sh.at[pl.ds(TAIL_0, TAIL_N)])

    plsc.subcore_barrier()

    def orow(i, carry):
        buf_ones[i, :] = onehot
        return carry

    lax.fori_loop(0, CH, orow, 0)

    BUFS0 = (buf_a0, buf_b0, buf_c0, sa0, sb0, sc0, ss0, so0)
    BUFS1 = (buf_a1, buf_b1, buf_c1, sa1, sb1, sc1, ss1, so1)

    def wait_scatters(bufs):
        ba, bb, bc, sa, sb, sc_, ss, so = bufs
        pltpu.make_async_copy(bc, agg_sh.at[idxd_buf.at[0]], ss).wait()
        pltpu.make_async_copy(buf_ones, deg_sh.at[idxd_buf.at[0]], so).wait()

    def fire(gbase, j, bufs):
        ba, bb, bc, sa, sb, sc_, ss, so = bufs

        @pl.when(j >= 2)
        def _():
            wait_scatters(bufs)

        pltpu.async_copy(a_hbm.at[idxs_buf.at[j]], ba, sa)
        pltpu.async_copy(b_hbm.at[idxd_buf.at[j]], bb, sb)
        pltpu.async_copy(c_hbm.at[pl.ds(gbase + j * CH, CH)], bc, sc_)

    def process(gbase, j, bufs):
        ba, bb, bc, sa, sb, sc_, ss, so = bufs
        sl = pl.ds(gbase + j * CH, CH)
        pltpu.make_async_copy(a_hbm.at[idxs_buf.at[j]], ba, sa).wait()
        pltpu.make_async_copy(b_hbm.at[idxd_buf.at[j]], bb, sb).wait()
        pltpu.make_async_copy(c_hbm.at[sl], bc, sc_).wait()

        def row(i, carry2):
            for k in range(D // 16):
                ks = pl.ds(k * 16, 16)
                x = ba[i, ks] + bb[i, ks] + bc[i, ks]
                bc[i, ks] = jnp.where(x >= 0, x, 0.01 * x)
            return carry2

        lax.fori_loop(0, CH, row, 0)
        pltpu.async_copy(bc, agg_sh.at[idxd_buf.at[j]], ss, add=True)
        pltpu.async_copy(buf_ones, deg_sh.at[idxd_buf.at[j]], so, add=True)
        pltpu.sync_copy(bc, fnew_hbm.at[sl])

    def group(g, carry):
        @pl.when(g > 0)
        def _():
            wait_scatters(BUFS0)
            wait_scatters(BUFS1)

        gsl = pl.ds(g * G, G)
        pltpu.sync_copy(src3.at[wid, gsl], idxs_buf)
        pltpu.sync_copy(dst3.at[wid, gsl], idxd_buf)
        gbase = wid * PER_W + g * (G * CH)
        fire(gbase, 0, BUFS0)

        def pair(t, carry2):
            fire(gbase, 2 * t + 1, BUFS1)
            process(gbase, 2 * t, BUFS0)

            @pl.when(t + 1 < HALF)
            def _():
                fire(gbase, 2 * t + 2, BUFS0)

            process(gbase, 2 * t + 1, BUFS1)
            return carry2

        lax.fori_loop(0, HALF, pair, 0)
        return carry

    lax.fori_loop(0, NGRP, group, 0)
    wait_scatters(BUFS0)
    wait_scatters(BUFS1)
    plsc.subcore_barrier()

    for t in range(NCPY):
        off = s * ROWS_S + t * CPY
        pltpu.sync_copy(agg_sh.at[pl.ds(off, CPY)], cbuf)
        pltpu.sync_copy(cbuf, aggp_hbm.at[c, pl.ds(off, CPY)])
        pltpu.sync_copy(deg_sh.at[pl.ds(off, CPY)], dbuf)
        pltpu.sync_copy(dbuf, degp_hbm.at[c, pl.ds(off, CPY)])

    @pl.when(s == NS - 1)
    def _():
        tl = pl.ds(TAIL_0, TAIL_N)
        pltpu.sync_copy(agg_sh.at[tl], cbuf.at[pl.ds(0, TAIL_N)])
        pltpu.sync_copy(cbuf.at[pl.ds(0, TAIL_N)], aggp_hbm.at[c, tl])
        pltpu.sync_copy(deg_sh.at[tl], dbuf.at[pl.ds(0, TAIL_N)])
        pltpu.sync_copy(dbuf.at[pl.ds(0, TAIL_N)], degp_hbm.at[c, tl])


def _sc_call(a_t, b_t, c_t, src, dst):
    mesh = plsc.VectorSubcoreMesh(core_axis_name="c", subcore_axis_name="s")
    src3 = src.reshape(NW, NCHUNK, CH)
    dst3 = dst.reshape(NW, NCHUNK, CH)
    fn = pl.kernel(
        _sc_body,
        out_type=[jax.ShapeDtypeStruct((E, D), jnp.float32),
                  jax.ShapeDtypeStruct((NC, N, D), jnp.float32),
                  jax.ShapeDtypeStruct((NC, N, 16), jnp.float32)],
        mesh=mesh,
        compiler_params=pltpu.CompilerParams(use_tc_tiling_on_sc=False),
        scratch_types=[
            pltpu.VMEM((G, CH), jnp.int32),
            pltpu.VMEM((G, CH), jnp.int32),
            pltpu.VMEM((CH, D), jnp.float32),
            pltpu.VMEM((CH, D), jnp.float32),
            pltpu.VMEM((CH, D), jnp.float32),
            pltpu.VMEM((CH, D), jnp.float32),
            pltpu.VMEM((CH, D), jnp.float32),
            pltpu.VMEM((CH, D), jnp.float32),
            pltpu.VMEM((CH, 16), jnp.float32),
            pltpu.VMEM_SHARED((N, D), jnp.float32),
            pltpu.VMEM_SHARED((N, 16), jnp.float32),
            pltpu.SemaphoreType.DMA,
            pltpu.SemaphoreType.DMA,
            pltpu.SemaphoreType.DMA,
            pltpu.SemaphoreType.DMA,
            pltpu.SemaphoreType.DMA,
            pltpu.SemaphoreType.DMA,
            pltpu.SemaphoreType.DMA,
            pltpu.SemaphoreType.DMA,
            pltpu.SemaphoreType.DMA,
            pltpu.SemaphoreType.DMA,
        ],
    )
    return fn(a_t, b_t, c_t, src3, dst3)


def _eres_body(fn_in, ef, out):
    out[...] = fn_in[...] + ef[...]


def _eres_call(f_new, ef):
    BE = 4000
    row = lambda i: (i, 0)
    return pl.pallas_call(
        _eres_body,
        grid=(E // BE,),
        in_specs=[pl.BlockSpec((BE, D), row), pl.BlockSpec((BE, D), row)],
        out_specs=pl.BlockSpec((BE, D), row),
        out_shape=jax.ShapeDtypeStruct((E, D), jnp.float32),
    )(f_new, ef)


def _node_body(h, u, aggp, degp, nf, gf, v1, v2, v3, bnu,
               nout, g8, acc):
    i = pl.program_id(0)

    @pl.when(i == 0)
    def _():
        acc[...] = jnp.zeros_like(acc)

    agg = aggp[0] + aggp[1]
    deg = jnp.maximum(degp[0, :, 0] + degp[1, :, 0], 1.0)
    hf = agg / deg[:, None]
    node_new = _act(jnp.dot(h[...], v1[...], preferred_element_type=jnp.float32)
                    + jnp.dot(hf, v2[...], preferred_element_type=jnp.float32)
                    + jnp.dot(u[...], v3[...], preferred_element_type=jnp.float32)
                    + bnu[...])
    nout[...] = node_new + nf[...]
    acc[0:1, :] += jnp.sum(node_new, axis=0, keepdims=True)
    acc[1:2, :] += jnp.sum(agg, axis=0, keepdims=True)
    acc[2:3, :] += jnp.sum(gf[...], axis=0, keepdims=True)
    npool = acc[0:1, :] / N
    epool = acc[1:2, :] / E
    gpool = acc[2:3, :] / N
    g = _act(jnp.dot(npool, v1[...], preferred_element_type=jnp.float32)
             + jnp.dot(epool, v2[...], preferred_element_type=jnp.float32)
             + jnp.dot(gpool, v3[...], preferred_element_type=jnp.float32)
             + bnu[...])
    g8[...] = jnp.broadcast_to(g, (8, D))


def _node_call(h, u, aggp, degp, nf, gf, v1, v2, v3, bnu):
    BN = 1000
    grid = (N // BN,)
    row = lambda i: (i, 0)
    full = lambda i: (0, 0)
    return pl.pallas_call(
        _node_body,
        grid=grid,
        in_specs=[pl.BlockSpec((BN, D), row), pl.BlockSpec((BN, D), row),
                  pl.BlockSpec((NC, BN, D), lambda i: (0, i, 0)),
                  pl.BlockSpec((NC, BN, 16), lambda i: (0, i, 0)),
                  pl.BlockSpec((BN, D), row), pl.BlockSpec((BN, D), row),
                  pl.BlockSpec((D, D), full), pl.BlockSpec((D, D), full),
                  pl.BlockSpec((D, D), full), pl.BlockSpec((1, D), full)],
        out_specs=[pl.BlockSpec((BN, D), row), pl.BlockSpec((8, D), full)],
        out_shape=[jax.ShapeDtypeStruct((N, D), jnp.float32),
                   jax.ShapeDtypeStruct((8, D), jnp.float32)],
        scratch_shapes=[pltpu.VMEM((8, D), jnp.float32)],
    )(h, u, aggp, degp, nf, gf, v1, v2, v3, bnu)


def _gout_body(gf, g8, out):
    out[...] = gf[...] + g8[0:1, :]


def _gout_call(gf, g8):
    BN = 2000
    return pl.pallas_call(
        _gout_body,
        grid=(N // BN,),
        in_specs=[pl.BlockSpec((BN, D), lambda i: (i, 0)),
                  pl.BlockSpec((8, D), lambda i: (0, 0))],
        out_specs=pl.BlockSpec((BN, D), lambda i: (i, 0)),
        out_shape=jax.ShapeDtypeStruct((N, D), jnp.float32),
    )(gf, g8)


def kernel(node_feats, edge_feats, graph_feats, edge_index,
           W_n, b_n, W_e, b_e, W_g, b_g, W_eu, b_eu, W_nu, b_nu):
    src = edge_index[0].astype(jnp.int32)
    dst = edge_index[1].astype(jnp.int32)
    w1, w2, w3, w4 = W_eu[0:D], W_eu[D:2 * D], W_eu[2 * D:3 * D], W_eu[3 * D:4 * D]
    v1, v2, v3 = W_nu[0:D], W_nu[D:2 * D], W_nu[2 * D:3 * D]
    bn = b_n.reshape(1, D)
    be = b_e.reshape(1, D)
    bg = b_g.reshape(1, D)
    beu = b_eu.reshape(1, D)
    bnu = b_nu.reshape(1, D)

    h, u, a_t, b_t = _pre_call(node_feats, graph_feats, W_n, bn, W_g, bg, w1, w2, w4)
    c_t = _edgec_call(edge_feats, W_e, be, w3, beu)
    f_new, aggp, degp = _sc_call(a_t, b_t, c_t, src, dst)
    edge_new = _eres_call(f_new, edge_feats)
    node_new, g8 = _node_call(h, u, aggp, degp, node_feats, graph_feats, v1, v2, v3, bnu)
    g_new = _gout_call(graph_feats, g8)
    return (node_new, edge_new, g_new)
